# baseline (speedup 1.0000x reference)
import functools

import jax
import jax.numpy as jnp
from jax import lax
from jax.experimental import pallas as pl
from jax.experimental.pallas import tpu as pltpu
from jax.experimental.pallas import tpu_sc as plsc

N_NODES = 10000
N_EDGES = 320000
D = 128

NP = 10240
CH = 128
NW = 32
TPW = 80
NCHUNK = NW * TPW
EPAD = NCHUNK * CH
ROWS_PER_TILE = NP // 16

_MESH = plsc.VectorSubcoreMesh(core_axis_name="c", subcore_axis_name="s")

NBUF = 2
G = 16
NG = TPW // G


def _fill_1d(ref, n16, val):
    def body(j, _):
        ref[pl.ds(j * 16, 16)] = jnp.full((16,), val, ref.dtype)
        return 0
    lax.fori_loop(0, n16, body, 0)


def _sc_segsum_body(with_deg, z_hbm, edges_hbm, acc_hbm, deg_hbm,
                    acc_sh, deg_sh, src_g, dst_g, rows_b, ones_b,
                    zacc_b, zdeg_b, s0, s1, sd, si):
    c = lax.axis_index("c")
    s = lax.axis_index("s")
    w = c * 16 + s
    sems = (s0, s1)

    start = w * TPW

    def load_group(g):
        gb = g % 2
        pltpu.sync_copy(edges_hbm.at[0, pl.ds(start + g * G, G)],
                        src_g.at[gb])
        pltpu.sync_copy(edges_hbm.at[1, pl.ds(start + g * G, G)],
                        dst_g.at[gb])

    def load_group_async(g):
        gb = g % 2
        pltpu.async_copy(edges_hbm.at[0, pl.ds(start + g * G, G)],
                         src_g.at[gb], si)
        pltpu.async_copy(edges_hbm.at[1, pl.ds(start + g * G, G)],
                         dst_g.at[gb], si)

    def wait_group(g):
        gb = g % 2
        pltpu.make_async_copy(edges_hbm.at[0, pl.ds(start + g * G, G)],
                              src_g.at[gb], si).wait()
        pltpu.make_async_copy(edges_hbm.at[1, pl.ds(start + g * G, G)],
                              dst_g.at[gb], si).wait()

    def issue(g, l, buf):
        pltpu.async_copy(z_hbm.at[src_g.at[g % 2, l]], rows_b.at[buf],
                         sems[buf], priority=1)

    def wait_deg():
        pltpu.make_async_copy(ones_b, deg_sh.at[dst_g.at[0, 0]],
                              sd).wait()

    def drain_scatter(g, l, buf, first=False):
        pltpu.make_async_copy(z_hbm.at[src_g.at[g % 2, l]], rows_b.at[buf],
                              sems[buf]).wait()
        if with_deg:
            if not first:
                wait_deg()
            pltpu.async_copy(ones_b, deg_sh.at[dst_g.at[g % 2, l]], sd,
                             add=True)
        pltpu.sync_copy(rows_b.at[buf], acc_sh.at[dst_g.at[g % 2, l]],
                        add=True)

    load_group(0)
    issue(0, 0, 0)

    _fill_1d(ones_b, CH // 16, 1.0)

    def zr(i, _):
        def inner(j, _):
            zacc_b[i, pl.ds(j * 16, 16)] = jnp.zeros((16,), jnp.float32)
            return 0
        lax.fori_loop(0, 8, inner, 0)
        return 0
    lax.fori_loop(0, 32, zr, 0)
    _fill_1d(zdeg_b, ROWS_PER_TILE // 16, 0.0)

    base = s * ROWS_PER_TILE
    for k in range(ROWS_PER_TILE // 32):
        pltpu.sync_copy(zacc_b, acc_sh.at[pl.ds(base + k * 32, 32)])
    pltpu.sync_copy(zdeg_b, deg_sh.at[pl.ds(base, ROWS_PER_TILE)])
    plsc.subcore_barrier()

    for g in range(NG):
        if with_deg and g > 0:
            wait_deg()
        if g + 1 < NG:
            load_group_async(g + 1)

        issue(g, 1, 1)
        drain_scatter(g, 0, 0, first=True)
        issue(g, 2, 0)
        drain_scatter(g, 1, 1)

        def pair_nocross(t, _, g=g):
            l0 = 2 * t
            issue(g, l0 + 1, 1)
            drain_scatter(g, l0, 0)
            issue(g, l0 + 2, 0)
            drain_scatter(g, l0 + 1, 1)
            return 0
        lax.fori_loop(1, G // 2 - 1, pair_nocross, 0)

        issue(g, G - 1, 1)
        drain_scatter(g, G - 2, 0)
        if g + 1 < NG:
            wait_group(g + 1)
            issue(g + 1, 0, 0)
        drain_scatter(g, G - 1, 1)

    if with_deg:
        wait_deg()
    plsc.subcore_barrier()

    pltpu.sync_copy(acc_sh.at[pl.ds(base, ROWS_PER_TILE)],
                    acc_hbm.at[c, pl.ds(base, ROWS_PER_TILE)])
    if with_deg:
        pltpu.sync_copy(deg_sh.at[pl.ds(base, ROWS_PER_TILE)],
                        deg_hbm.at[c, pl.ds(base, ROWS_PER_TILE)])


_sc_scratch = [
    pltpu.VMEM_SHARED((NP, D), jnp.float32),
    pltpu.VMEM_SHARED((NP,), jnp.float32),
    pltpu.VMEM((2, G, CH), jnp.int32),
    pltpu.VMEM((2, G, CH), jnp.int32),
    pltpu.VMEM((NBUF, CH, D), jnp.float32),
    pltpu.VMEM((CH,), jnp.float32),
    pltpu.VMEM((32, D), jnp.float32),
    pltpu.VMEM((ROWS_PER_TILE,), jnp.float32),
    pltpu.SemaphoreType.DMA,
    pltpu.SemaphoreType.DMA,
    pltpu.SemaphoreType.DMA,
    pltpu.SemaphoreType.DMA,
]

_sc_segsum_deg = pl.kernel(
    functools.partial(_sc_segsum_body, True),
    out_type=[jax.ShapeDtypeStruct((2, NP, D), jnp.float32),
              jax.ShapeDtypeStruct((2, NP), jnp.float32)],
    mesh=_MESH,
    scratch_types=_sc_scratch,
)

_sc_segsum_nodeg = pl.kernel(
    functools.partial(_sc_segsum_body, False),
    out_type=[jax.ShapeDtypeStruct((2, NP, D), jnp.float32),
              jax.ShapeDtypeStruct((2, NP), jnp.float32)],
    mesh=_MESH,
    scratch_types=_sc_scratch,
)


BLK = 2000
GRID = N_NODES // BLK


def _dot_t(x, w):
    return lax.dot_general(x, w, (((1,), (1,)), ((), ())),
                           preferred_element_type=jnp.float32)


def _mm2_body(x_ref, wl_ref, wr_ref, b_ref, z_ref, r_ref):
    x = x_ref[...]
    z_ref[...] = _dot_t(x, wl_ref[...])
    r_ref[...] = _dot_t(x, wr_ref[...]) + b_ref[...]


def _comb_mm_body(acc_ref, deg_ref, r_ref, wl_ref, wr_ref, b_ref,
                  z_ref, rr_ref):
    ssum = acc_ref[0] + acc_ref[1]
    d = deg_ref[0] + deg_ref[1]
    h = ssum / jnp.maximum(d, 1.0) + r_ref[...]
    z_ref[...] = _dot_t(h, wl_ref[...])
    rr_ref[...] = _dot_t(h, wr_ref[...]) + b_ref[...]


def _comb_body(acc_ref, deg_ref, r_ref, o_ref):
    ssum = acc_ref[0] + acc_ref[1]
    d = deg_ref[0] + deg_ref[1]
    o_ref[...] = ssum / jnp.maximum(d, 1.0) + r_ref[...]


_x_spec = pl.BlockSpec((BLK, D), lambda i: (i, 0))
_w_spec = pl.BlockSpec((D, D), lambda i: (0, 0))
_b_spec = pl.BlockSpec((1, D), lambda i: (0, 0))
_acc_spec = pl.BlockSpec((2, BLK, D), lambda i: (0, i, 0))
_deg_spec = pl.BlockSpec((2, BLK, 1), lambda i: (0, i, 0))

_mm2 = pl.pallas_call(
    _mm2_body,
    grid=(GRID,),
    in_specs=[_x_spec, _w_spec, _w_spec, _b_spec],
    out_specs=[_x_spec, _x_spec],
    out_shape=[jax.ShapeDtypeStruct((N_NODES, D), jnp.float32)] * 2,
)

_comb_mm = pl.pallas_call(
    _comb_mm_body,
    grid=(GRID,),
    in_specs=[_acc_spec, _deg_spec, _x_spec, _w_spec, _w_spec, _b_spec],
    out_specs=[_x_spec, _x_spec],
    out_shape=[jax.ShapeDtypeStruct((N_NODES, D), jnp.float32)] * 2,
)

_comb = pl.pallas_call(
    _comb_body,
    grid=(GRID,),
    in_specs=[_acc_spec, _deg_spec, _x_spec],
    out_specs=_x_spec,
    out_shape=jax.ShapeDtypeStruct((N_NODES, D), jnp.float32),
)


def kernel(nodes, edge_list, Wl1, bl1, Wr1, Wl2, bl2, Wr2):
    npad = EPAD - N_EDGES
    pad_src = (jnp.arange(npad, dtype=jnp.int32) * 7) % N_NODES
    pad_dst = N_NODES + (jnp.arange(npad, dtype=jnp.int32) % (NP - N_NODES))
    edges3 = jnp.concatenate(
        [edge_list, jnp.stack([pad_src, pad_dst])], axis=1
    ).reshape(2, NCHUNK, CH)
    b1 = bl1.reshape(1, D)
    b2 = bl2.reshape(1, D)

    z1, r1 = _mm2(nodes, Wl1, Wr1, b1)
    acc1, deg1 = _sc_segsum_deg(z1, edges3)
    deg3 = deg1.reshape(2, NP, 1)
    z2, r2 = _comb_mm(acc1, deg3, r1, Wl2, Wr2, b2)
    acc2, _unused = _sc_segsum_nodeg(z2, edges3)
    return _comb(acc2, deg3, r2)


# speedup vs baseline: 1.0199x; 1.0199x over previous
import functools

import jax
import jax.numpy as jnp
import numpy as np
from jax import lax
from jax.experimental import pallas as pl
from jax.experimental.pallas import tpu as pltpu
from jax.experimental.pallas import tpu_sc as plsc

N_NODES = 10000
N_EDGES = 320000
D = 128

NP = 10240
CH = 128
NW = 32
TPW = 80
NCHUNK = NW * TPW
EPAD = NCHUNK * CH
ROWS_PER_TILE = NP // 16

_MESH = plsc.VectorSubcoreMesh(core_axis_name="c", subcore_axis_name="s")

NBUF = 2
G = 16
NG = TPW // G


def _fill_1d(ref, n16, val):
    def body(j, _):
        ref[pl.ds(j * 16, 16)] = jnp.full((16,), val, ref.dtype)
        return 0
    lax.fori_loop(0, n16, body, 0)


def _sc_segsum_body(with_deg, z_hbm, edges_hbm, acc_hbm, deg_hbm,
                    acc_sh, deg_sh, src_g, dst_g, rows_b, ones_b,
                    zacc_b, zdeg_b, s0, s1, sd, si):
    c = lax.axis_index("c")
    s = lax.axis_index("s")
    w = c * 16 + s
    sems = (s0, s1)

    start = w * TPW

    def load_group(g):
        gb = g % 2
        pltpu.sync_copy(edges_hbm.at[0, pl.ds(start + g * G, G)],
                        src_g.at[gb])
        pltpu.sync_copy(edges_hbm.at[1, pl.ds(start + g * G, G)],
                        dst_g.at[gb])

    def load_group_async(g):
        gb = g % 2
        pltpu.async_copy(edges_hbm.at[0, pl.ds(start + g * G, G)],
                         src_g.at[gb], si)
        pltpu.async_copy(edges_hbm.at[1, pl.ds(start + g * G, G)],
                         dst_g.at[gb], si)

    def wait_group(g):
        gb = g % 2
        pltpu.make_async_copy(edges_hbm.at[0, pl.ds(start + g * G, G)],
                              src_g.at[gb], si).wait()
        pltpu.make_async_copy(edges_hbm.at[1, pl.ds(start + g * G, G)],
                              dst_g.at[gb], si).wait()

    def issue(g, l, buf):
        pltpu.async_copy(z_hbm.at[src_g.at[g % 2, l]], rows_b.at[buf],
                         sems[buf])

    def wait_deg():
        pltpu.make_async_copy(ones_b, deg_sh.at[dst_g.at[0, 0]],
                              sd).wait()

    def drain_scatter(g, l, buf, first=False):
        pltpu.make_async_copy(z_hbm.at[src_g.at[g % 2, l]], rows_b.at[buf],
                              sems[buf]).wait()
        if with_deg:
            if not first:
                wait_deg()
            pltpu.async_copy(ones_b, deg_sh.at[dst_g.at[g % 2, l]], sd,
                             add=True)
        pltpu.sync_copy(rows_b.at[buf], acc_sh.at[dst_g.at[g % 2, l]],
                        add=True)

    load_group(0)
    issue(0, 0, 0)

    _fill_1d(ones_b, CH // 16, 1.0)

    def zr(i, _):
        def inner(j, _):
            zacc_b[i, pl.ds(j * 16, 16)] = jnp.zeros((16,), jnp.float32)
            return 0
        lax.fori_loop(0, 8, inner, 0)
        return 0
    lax.fori_loop(0, 32, zr, 0)
    _fill_1d(zdeg_b, ROWS_PER_TILE // 16, 0.0)

    base = s * ROWS_PER_TILE
    for k in range(ROWS_PER_TILE // 32):
        pltpu.async_copy(zacc_b, acc_sh.at[pl.ds(base + k * 32, 32)], si)
    pltpu.sync_copy(zdeg_b, deg_sh.at[pl.ds(base, ROWS_PER_TILE)])
    for k in range(ROWS_PER_TILE // 32):
        pltpu.make_async_copy(zacc_b, acc_sh.at[pl.ds(base + k * 32, 32)],
                              si).wait()
    plsc.subcore_barrier()

    for g in range(NG):
        if with_deg and g > 0:
            wait_deg()
        if g + 1 < NG:
            load_group_async(g + 1)

        issue(g, 1, 1)
        drain_scatter(g, 0, 0, first=True)
        issue(g, 2, 0)
        drain_scatter(g, 1, 1)

        def pair_nocross(t, _, g=g):
            l0 = 2 * t
            issue(g, l0 + 1, 1)
            drain_scatter(g, l0, 0)
            issue(g, l0 + 2, 0)
            drain_scatter(g, l0 + 1, 1)
            return 0
        lax.fori_loop(1, G // 2 - 1, pair_nocross, 0)

        issue(g, G - 1, 1)
        drain_scatter(g, G - 2, 0)
        if g + 1 < NG:
            wait_group(g + 1)
            issue(g + 1, 0, 0)
        drain_scatter(g, G - 1, 1)

    if with_deg:
        wait_deg()
    plsc.subcore_barrier()

    pltpu.sync_copy(acc_sh.at[pl.ds(base, ROWS_PER_TILE)],
                    acc_hbm.at[c, pl.ds(base, ROWS_PER_TILE)])
    if with_deg:
        pltpu.sync_copy(deg_sh.at[pl.ds(base, ROWS_PER_TILE)],
                        deg_hbm.at[c, pl.ds(base, ROWS_PER_TILE)])


_sc_scratch = [
    pltpu.VMEM_SHARED((NP, D), jnp.float32),
    pltpu.VMEM_SHARED((NP,), jnp.float32),
    pltpu.VMEM((2, G, CH), jnp.int32),
    pltpu.VMEM((2, G, CH), jnp.int32),
    pltpu.VMEM((NBUF, CH, D), jnp.float32),
    pltpu.VMEM((CH,), jnp.float32),
    pltpu.VMEM((32, D), jnp.float32),
    pltpu.VMEM((ROWS_PER_TILE,), jnp.float32),
    pltpu.SemaphoreType.DMA,
    pltpu.SemaphoreType.DMA,
    pltpu.SemaphoreType.DMA,
    pltpu.SemaphoreType.DMA,
]

_sc_segsum_deg = pl.kernel(
    functools.partial(_sc_segsum_body, True),
    out_type=[jax.ShapeDtypeStruct((2, NP, D), jnp.float32),
              jax.ShapeDtypeStruct((2, NP), jnp.float32)],
    mesh=_MESH,
    scratch_types=_sc_scratch,
)

_sc_segsum_nodeg = pl.kernel(
    functools.partial(_sc_segsum_body, False),
    out_type=[jax.ShapeDtypeStruct((2, NP, D), jnp.float32),
              jax.ShapeDtypeStruct((2, NP), jnp.float32)],
    mesh=_MESH,
    scratch_types=_sc_scratch,
)


BLK = 2000
GRID = N_NODES // BLK


def _dot_t(x, w):
    return lax.dot_general(x, w, (((1,), (1,)), ((), ())),
                           preferred_element_type=jnp.float32)


def _mm2_body(x_ref, wl_ref, wr_ref, b_ref, z_ref, r_ref):
    x = x_ref[...]
    z_ref[...] = _dot_t(x, wl_ref[...])
    r_ref[...] = _dot_t(x, wr_ref[...]) + b_ref[...]


def _comb_mm_body(acc_ref, deg_ref, r_ref, wl_ref, wr_ref, b_ref,
                  z_ref, rr_ref):
    ssum = acc_ref[0] + acc_ref[1]
    d = deg_ref[0] + deg_ref[1]
    h = ssum / jnp.maximum(d, 1.0) + r_ref[...]
    z_ref[...] = _dot_t(h, wl_ref[...])
    rr_ref[...] = _dot_t(h, wr_ref[...]) + b_ref[...]


def _comb_body(acc_ref, deg_ref, r_ref, o_ref):
    ssum = acc_ref[0] + acc_ref[1]
    d = deg_ref[0] + deg_ref[1]
    o_ref[...] = ssum / jnp.maximum(d, 1.0) + r_ref[...]


_x_spec = pl.BlockSpec((BLK, D), lambda i: (i, 0))
_w_spec = pl.BlockSpec((D, D), lambda i: (0, 0))
_b_spec = pl.BlockSpec((1, D), lambda i: (0, 0))
_acc_spec = pl.BlockSpec((2, BLK, D), lambda i: (0, i, 0))
_deg_spec = pl.BlockSpec((2, BLK, 1), lambda i: (0, i, 0))

_mm2 = pl.pallas_call(
    _mm2_body,
    grid=(GRID,),
    in_specs=[_x_spec, _w_spec, _w_spec, _b_spec],
    out_specs=[_x_spec, _x_spec],
    out_shape=[jax.ShapeDtypeStruct((N_NODES, D), jnp.float32)] * 2,
)

_comb_mm = pl.pallas_call(
    _comb_mm_body,
    grid=(GRID,),
    in_specs=[_acc_spec, _deg_spec, _x_spec, _w_spec, _w_spec, _b_spec],
    out_specs=[_x_spec, _x_spec],
    out_shape=[jax.ShapeDtypeStruct((N_NODES, D), jnp.float32)] * 2,
)

_comb = pl.pallas_call(
    _comb_body,
    grid=(GRID,),
    in_specs=[_acc_spec, _deg_spec, _x_spec],
    out_specs=_x_spec,
    out_shape=jax.ShapeDtypeStruct((N_NODES, D), jnp.float32),
)


_NPAD = EPAD - N_EDGES
_PAD_EDGES = jnp.asarray(np.stack([
    (np.arange(_NPAD) * 7) % N_NODES,
    N_NODES + np.arange(_NPAD) % (NP - N_NODES),
]).astype(np.int32))


def kernel(nodes, edge_list, Wl1, bl1, Wr1, Wl2, bl2, Wr2):
    edges3 = jnp.concatenate([edge_list, _PAD_EDGES],
                             axis=1).reshape(2, NCHUNK, CH)
    b1 = bl1.reshape(1, D)
    b2 = bl2.reshape(1, D)

    z1, r1 = _mm2(nodes, Wl1, Wr1, b1)
    acc1, deg1 = _sc_segsum_deg(z1, edges3)
    deg3 = deg1.reshape(2, NP, 1)
    z2, r2 = _comb_mm(acc1, deg3, r1, Wl2, Wr2, b2)
    acc2, _unused = _sc_segsum_nodeg(z2, edges3)
    return _comb(acc2, deg3, r2)
